# baseline (speedup 1.0000x reference)
import jax
import jax.numpy as jnp
from jax import lax
from jax.experimental import pallas as pl
from jax.experimental.pallas import tpu as pltpu
from jax.experimental.pallas import tpu_sc as plsc

_BINS = 32
_EMBED = 64
_NUM_EMB = 2 * _BINS + 2
_B = 64
_L = 4096
_NC = 2
_NS = 16
_NW = _NC * _NS
_ROWS_PER_W = _B // _NW
_TOK_PER_W = _ROWS_PER_W * _L
_CHUNK = 512
_NCHUNK = _TOK_PER_W // _CHUNK
_LANES = 16


def _body(query_hbm, key_hbm, table_hbm, out_hbm, query_v, table_v, keys_v,
          idx_v, rows_v, sem_o):
    wid = lax.axis_index("s") * _NC + lax.axis_index("c")
    t0 = wid * _TOK_PER_W
    pltpu.sync_copy(query_hbm, query_v)
    pltpu.sync_copy(table_hbm, table_v)
    pltpu.sync_copy(key_hbm.at[pl.ds(t0, _TOK_PER_W)], keys_v)

    base = wid * _ROWS_PER_W
    vbase = (base // _LANES) * _LANES
    qvec = query_v[pl.ds(vbase, _LANES)]
    for r in range(_ROWS_PER_W):
        lane = base + r - vbase
        q = qvec.at[jnp.full((_LANES,), lane, jnp.int32)].get(
            mode="promise_in_bounds")

        def vec(i, _, q=q, off=r * _L):
            kv = keys_v[pl.ds(off + i * _LANES, _LANES)]
            d = jnp.clip(kv - q, -_BINS, _BINS) + (_BINS + 1)
            idx_v[pl.ds(off + i * _LANES, _LANES)] = d
            return 0

        lax.fori_loop(0, _L // _LANES, vec, 0)

    col = lax.iota(jnp.int32, 16)
    out_copies = []
    for c in range(_NCHUNK):
        p = c % 2
        if c >= 2:
            out_copies[c - 2].wait()

        def group(g, _, c=c, p=p):
            r_vec = idx_v[pl.ds(c * _CHUNK + g * _LANES, _LANES)]
            gbase = g * (_LANES * _EMBED)
            for k in range(_LANES):
                row = r_vec.at[jnp.full((_LANES,), k, jnp.int32)].get(
                    mode="promise_in_bounds")
                for j in range(_EMBED // _LANES):
                    v = plsc.load_gather(table_v, [row, col + j * _LANES])
                    rows_v[p, pl.ds(gbase + k * _EMBED + j * _LANES,
                                    _LANES)] = v
            return 0

        lax.fori_loop(0, _CHUNK // _LANES, group, 0)
        out_copies.append(pltpu.async_copy(
            rows_v.at[p],
            out_hbm.at[pl.ds((t0 + c * _CHUNK) * _EMBED, _CHUNK * _EMBED)],
            sem_o))
    out_copies[-2].wait()
    out_copies[-1].wait()


@jax.jit
def kernel(query_residue_index, key_residue_index, weight):
    mesh = plsc.VectorSubcoreMesh(core_axis_name="c", subcore_axis_name="s")
    run = pl.kernel(
        _body,
        out_type=jax.ShapeDtypeStruct((_B * _L * _EMBED,), jnp.float32),
        mesh=mesh,
        compiler_params=pltpu.CompilerParams(
            use_tc_tiling_on_sc=False, needs_layout_passes=False),
        scratch_types=[
            pltpu.VMEM((_B,), jnp.int32),
            pltpu.VMEM((_NUM_EMB, _EMBED), jnp.float32),
            pltpu.VMEM((_TOK_PER_W,), jnp.int32),
            pltpu.VMEM((_TOK_PER_W,), jnp.int32),
            pltpu.VMEM((2, _CHUNK * _EMBED), jnp.float32),
            pltpu.SemaphoreType.DMA,
        ],
    )
    out = run(query_residue_index.reshape(-1),
              key_residue_index.reshape(-1), weight)
    return out.reshape(_B, _L, _EMBED)


# speedup vs baseline: 1.3898x; 1.3898x over previous
import jax
import jax.numpy as jnp
from jax import lax
from jax.experimental import pallas as pl
from jax.experimental.pallas import tpu as pltpu
from jax.experimental.pallas import tpu_sc as plsc

_BINS = 32
_EMBED = 64
_NUM_EMB = 2 * _BINS + 2
_B = 64
_L = 4096
_NC = 2
_NS = 16
_NW = _NC * _NS
_ROWS_PER_W = _B // _NW
_TOK_PER_W = _ROWS_PER_W * _L
_CHUNK = 512
_NCHUNK = _TOK_PER_W // _CHUNK
_LANES = 16


def _body(query_hbm, key_hbm, table_hbm, out_hbm, query_v, table_v, keys_v,
          idx_v, rows_v, sem_o):
    wid = lax.axis_index("s") * _NC + lax.axis_index("c")
    t0 = wid * _TOK_PER_W
    pltpu.sync_copy(query_hbm, query_v)
    pltpu.sync_copy(table_hbm, table_v)
    pltpu.sync_copy(key_hbm.at[pl.ds(t0, _TOK_PER_W)], keys_v)

    base = wid * _ROWS_PER_W
    vbase = (base // _LANES) * _LANES
    qvec = query_v[pl.ds(vbase, _LANES)]
    for r in range(_ROWS_PER_W):
        lane = base + r - vbase
        q = qvec.at[jnp.full((_LANES,), lane, jnp.int32)].get(
            mode="promise_in_bounds")

        @plsc.parallel_loop(0, _L // _LANES, unroll=4)
        def _vec(i, q=q, off=r * _L):
            kv = keys_v[pl.ds(off + i * _LANES, _LANES)]
            d = jnp.clip(kv - q, -_BINS, _BINS) + (_BINS + 1)
            idx_v[pl.ds(off + i * _LANES, _LANES)] = d

    col = lax.iota(jnp.int32, 16)
    cfloats = _CHUNK * _EMBED

    def outer(oc, _):
        for p in range(2):
            c = oc * 2 + p

            @pl.when(oc > 0)
            def _wait_prev(p=p):
                pltpu.make_async_copy(
                    rows_v.at[p], out_hbm.at[pl.ds(0, cfloats)], sem_o).wait()

            @plsc.parallel_loop(0, _CHUNK // _LANES, unroll=2)
            def _group(g, c=c, p=p):
                r_vec = idx_v[pl.ds(c * _CHUNK + g * _LANES, _LANES)]
                gbase = g * (_LANES * _EMBED)
                for k in range(_LANES):
                    row = r_vec.at[jnp.full((_LANES,), k, jnp.int32)].get(
                        mode="promise_in_bounds")
                    for j in range(_EMBED // _LANES):
                        v = plsc.load_gather(table_v, [row, col + j * _LANES])
                        rows_v[p, pl.ds(gbase + k * _EMBED + j * _LANES,
                                        _LANES)] = v

            pltpu.async_copy(
                rows_v.at[p],
                out_hbm.at[pl.ds((t0 + c * _CHUNK) * _EMBED, cfloats)],
                sem_o)
        return 0

    lax.fori_loop(0, _NCHUNK // 2, outer, 0)
    for p in range(2):
        pltpu.make_async_copy(
            rows_v.at[p], out_hbm.at[pl.ds(0, cfloats)], sem_o).wait()


@jax.jit
def kernel(query_residue_index, key_residue_index, weight):
    mesh = plsc.VectorSubcoreMesh(core_axis_name="c", subcore_axis_name="s")
    run = pl.kernel(
        _body,
        out_type=jax.ShapeDtypeStruct((_B * _L * _EMBED,), jnp.float32),
        mesh=mesh,
        compiler_params=pltpu.CompilerParams(
            use_tc_tiling_on_sc=False, needs_layout_passes=False),
        scratch_types=[
            pltpu.VMEM((_B,), jnp.int32),
            pltpu.VMEM((_NUM_EMB, _EMBED), jnp.float32),
            pltpu.VMEM((_TOK_PER_W,), jnp.int32),
            pltpu.VMEM((_TOK_PER_W,), jnp.int32),
            pltpu.VMEM((2, _CHUNK * _EMBED), jnp.float32),
            pltpu.SemaphoreType.DMA,
        ],
    )
    out = run(query_residue_index.reshape(-1),
              key_residue_index.reshape(-1), weight)
    return out.reshape(_B, _L, _EMBED)
